# baseline (speedup 1.0000x reference)
import functools

import jax
import jax.numpy as jnp
from jax import lax
from jax.experimental import pallas as pl
from jax.experimental.pallas import tpu as pltpu
from jax.experimental.pallas import tpu_sc as plsc

NC = 2
NS = 16
K = 80
NZ = 125


def _fill(ref, nrows, ncols, value):
    npc = ncols // 16
    vec = jnp.full((16,), value, jnp.float32)

    def body(t, carry):
        r = t // npc
        j = t % npc
        ref[r, pl.ds(j * 16, 16)] = vec
        return carry

    lax.fori_loop(0, nrows * npc, body, 0)


def _make_deg(n, e):
    ept = e // (NC * NS)
    ni = ept // K
    npt = n // NS
    mesh = plsc.VectorSubcoreMesh(core_axis_name="c", subcore_axis_name="s", num_cores=NC, num_subcores=NS)

    @functools.partial(
        pl.kernel,
        out_type=jax.ShapeDtypeStruct((NC, NS, npt, 16), jnp.float32),
        mesh=mesh,
        scratch_types=[
            pltpu.VMEM((ni, K), jnp.int32),
            pltpu.VMEM((K, 16), jnp.float32),
            pltpu.VMEM((NZ, 16), jnp.float32),
            pltpu.SemaphoreType.DMA,
            pltpu.VMEM_SHARED((n, 16), jnp.float32),
        ],
        compiler_params=pltpu.CompilerParams(use_tc_tiling_on_sc=False),
    )
    def deg_kernel(dst_hbm, out_hbm, idx_v, ones_v, z_v, sem, acc):
        cid = lax.axis_index("c")
        sid = lax.axis_index("s")
        _fill(z_v, NZ, 16, 0.0)
        _fill(ones_v, K, 16, 1.0)
        for j in range(npt // NZ):
            pltpu.sync_copy(z_v, acc.at[pl.ds(sid * npt + j * NZ, NZ)])
        pltpu.sync_copy(dst_hbm.at[cid * NS + sid], idx_v)
        plsc.subcore_barrier()

        kf = 25
        assert ni % kf == 0

        def gbody(g, carry):
            def fire(i, c):
                pltpu.async_copy(ones_v, acc.at[idx_v.at[g * kf + i]],
                                 sem, add=True)
                return c
            lax.fori_loop(0, kf, fire, 0)

            def drain(i, c):
                pltpu.make_async_copy(ones_v, acc.at[idx_v.at[g * kf + i]],
                                      sem).wait()
                return c
            lax.fori_loop(0, kf, drain, 0)
            return carry

        lax.fori_loop(0, ni // kf, gbody, 0)
        plsc.subcore_barrier()
        pltpu.sync_copy(acc.at[pl.ds(sid * npt, npt)], out_hbm.at[cid, sid])

    return deg_kernel


def _make_scatter(n, e, dh):
    kc = 128
    nr = e // kc
    npt = n // NS
    q = nr // NS
    assert q * NS + 4 == nr and (q + 1) % 4 == 1
    mesh = plsc.VectorSubcoreMesh(core_axis_name="c", subcore_axis_name="s", num_cores=NC, num_subcores=NS)

    nb = 4
    nloops = q // nb
    assert q % nb == 0

    @functools.partial(
        pl.kernel,
        out_type=jax.ShapeDtypeStruct((NC, NS, npt, dh), jnp.float32),
        mesh=mesh,
        scratch_types=[
            pltpu.VMEM((q + 1, kc), jnp.int32),
            pltpu.VMEM((q + 1, kc), jnp.int32),
            [pltpu.VMEM((kc, dh), jnp.float32) for _ in range(nb)],
            pltpu.VMEM((NZ, dh), jnp.float32),
            [pltpu.SemaphoreType.DMA for _ in range(nb)],
            pltpu.VMEM_SHARED((n, dh), jnp.float32),
        ],
        compiler_params=pltpu.CompilerParams(use_tc_tiling_on_sc=False),
    )
    def scat_kernel(g_hbm, src_hbm, dst_hbm, out_hbm,
                    si_v, di_v, rows, z_v, sems, acc):
        cid = lax.axis_index("c")
        sid = lax.axis_index("s")
        _fill(z_v, NZ, dh, 0.0)
        for j in range(npt // NZ):
            pltpu.sync_copy(z_v, acc.at[pl.ds(sid * npt + j * NZ, NZ)])
        r0 = sid * q + jnp.maximum(sid - (NS - 4), 0)
        nt = jnp.where(sid >= NS - 4, q + 1, q)
        pltpu.sync_copy(src_hbm.at[pl.ds(r0, q + 1)], si_v)
        pltpu.sync_copy(dst_hbm.at[pl.ds(r0, q + 1)], di_v)
        plsc.subcore_barrier()

        for b in range(nb):
            pltpu.async_copy(g_hbm.at[cid].at[si_v.at[b]], rows[b], sems[b])

        def ebody(jj, carry):
            for b in range(nb):
                i = jj * nb + b
                pltpu.make_async_copy(g_hbm.at[cid].at[si_v.at[i]],
                                      rows[b], sems[b]).wait()
                pltpu.sync_copy(rows[b], acc.at[di_v.at[i]], add=True)

                @pl.when(i + nb < nt)
                def _():
                    pltpu.async_copy(g_hbm.at[cid].at[si_v.at[i + nb]],
                                     rows[b], sems[b])
            return carry

        lax.fori_loop(0, nloops, ebody, 0)

        @pl.when(nt > q)
        def _():
            pltpu.make_async_copy(g_hbm.at[cid].at[si_v.at[q]],
                                  rows[0], sems[0]).wait()
            pltpu.sync_copy(rows[0], acc.at[di_v.at[q]], add=True)

        plsc.subcore_barrier()
        pltpu.sync_copy(acc.at[pl.ds(sid * npt, npt)], out_hbm.at[cid, sid])

    return scat_kernel


def _tc_matmul(x_ref, w_ref, o_ref):
    o_ref[...] = jnp.dot(x_ref[...], w_ref[...],
                         preferred_element_type=jnp.float32)


def _make_tc_layer1(dh):
    def _tc_layer1(xw_ref, deg_ref, o_ref):
        dinv = lax.rsqrt(deg_ref[...] + 1.0)
        xw = xw_ref[...] * dinv
        o_ref[0] = xw[:, :dh]
        o_ref[1] = xw[:, dh:]

    return _tc_layer1


def _make_tc_layer2(d2h):
    def _tc_layer2(p_ref, g_ref, deg_ref, w_ref, o_ref):
        dinv = lax.rsqrt(deg_ref[...] + 1.0)
        s = p_ref[...] + g_ref[...]
        h = jnp.concatenate([s[0], s[1]], axis=1) * dinv
        h = jnp.maximum(h, 0.0)
        g2 = jnp.dot(h, w_ref[...], preferred_element_type=jnp.float32) * dinv
        o_ref[0] = g2[:, :d2h]
        o_ref[1] = g2[:, d2h:]

    return _tc_layer2


def _make_tc_final(d_out):
    def _tc_final(p_ref, g_ref, deg_ref, o_ref):
        dinv = lax.rsqrt(deg_ref[...] + 1.0)
        s = p_ref[...] + g_ref[...]
        out = jnp.concatenate([s[0], s[1]], axis=1) * dinv
        o_ref[...] = out[:, :d_out]

    return _tc_final


def kernel(x, edge_index, y, W1, W2):
    n, _ = x.shape
    e = edge_index.shape[1]
    d_hid = W1.shape[1]
    d_out = W2.shape[1]
    d2p = 64
    dh = d_hid // 2
    d2h = d2p // 2

    ept = e // (NC * NS)
    srcd = edge_index[0].reshape(e // 128, 128)
    dstd = edge_index[1].reshape(e // 128, 128)
    dst_deg = edge_index[1].reshape(NC * NS, ept // K, K)

    xw = pl.pallas_call(
        _tc_matmul,
        out_shape=jax.ShapeDtypeStruct((n, d_hid), jnp.float32),
    )(x, W1)

    degp = _make_deg(n, e)(dst_deg).reshape(NC, n, 16)
    deg = (degp[0] + degp[1])[:, :1]

    g1 = pl.pallas_call(
        _make_tc_layer1(dh),
        out_shape=jax.ShapeDtypeStruct((NC, n, dh), jnp.float32),
    )(xw, deg)

    p1 = _make_scatter(n, e, dh)(g1, srcd, dstd).reshape(NC, n, dh)

    w2p = jnp.pad(W2, ((0, 0), (0, d2p - d_out)))
    g2 = pl.pallas_call(
        _make_tc_layer2(d2h),
        out_shape=jax.ShapeDtypeStruct((NC, n, d2h), jnp.float32),
    )(p1, g1, deg, w2p)

    p2 = _make_scatter(n, e, d2h)(g2, srcd, dstd).reshape(NC, n, d2h)

    out = pl.pallas_call(
        _make_tc_final(d_out),
        out_shape=jax.ShapeDtypeStruct((n, d_out), jnp.float32),
    )(p2, g2, deg)
    return out


# speedup vs baseline: 1.0047x; 1.0047x over previous
import functools

import jax
import jax.numpy as jnp
from jax import lax
from jax.experimental import pallas as pl
from jax.experimental.pallas import tpu as pltpu
from jax.experimental.pallas import tpu_sc as plsc

NC = 2
NS = 16
K = 80
NZ = 125


def _fill(ref, nrows, ncols, value):
    npc = ncols // 16
    vec = jnp.full((16,), value, jnp.float32)

    def body(t, carry):
        r = t // npc
        j = t % npc
        ref[r, pl.ds(j * 16, 16)] = vec
        return carry

    lax.fori_loop(0, nrows * npc, body, 0)


def _make_deg(n, e):
    ept = e // (NC * NS)
    ni = ept // K
    npt = n // NS
    mesh = plsc.VectorSubcoreMesh(core_axis_name="c", subcore_axis_name="s", num_cores=NC, num_subcores=NS)

    @functools.partial(
        pl.kernel,
        out_type=jax.ShapeDtypeStruct((NC, NS, npt, 16), jnp.float32),
        mesh=mesh,
        scratch_types=[
            pltpu.VMEM((ni, K), jnp.int32),
            pltpu.VMEM((K, 16), jnp.float32),
            pltpu.VMEM((NZ, 16), jnp.float32),
            pltpu.SemaphoreType.DMA,
            pltpu.VMEM_SHARED((n, 16), jnp.float32),
        ],
        compiler_params=pltpu.CompilerParams(use_tc_tiling_on_sc=False),
    )
    def deg_kernel(dst_hbm, out_hbm, idx_v, ones_v, z_v, sem, acc):
        cid = lax.axis_index("c")
        sid = lax.axis_index("s")
        _fill(z_v, NZ, 16, 0.0)
        _fill(ones_v, K, 16, 1.0)
        for j in range(npt // NZ):
            pltpu.sync_copy(z_v, acc.at[pl.ds(sid * npt + j * NZ, NZ)])
        pltpu.sync_copy(dst_hbm.at[cid * NS + sid], idx_v)
        plsc.subcore_barrier()

        kf = 25
        assert ni % kf == 0

        def gbody(g, carry):
            def fire(i, c):
                pltpu.async_copy(ones_v, acc.at[idx_v.at[g * kf + i]],
                                 sem, add=True)
                return c
            lax.fori_loop(0, kf, fire, 0)

            def drain(i, c):
                pltpu.make_async_copy(ones_v, acc.at[idx_v.at[g * kf + i]],
                                      sem).wait()
                return c
            lax.fori_loop(0, kf, drain, 0)
            return carry

        lax.fori_loop(0, ni // kf, gbody, 0)
        plsc.subcore_barrier()
        pltpu.sync_copy(acc.at[pl.ds(sid * npt, npt)], out_hbm.at[cid, sid])

    return deg_kernel


def _make_scatter(n, e, dh):
    kc = 128
    nr = e // kc
    npt = n // NS
    q = nr // NS
    assert q * NS + 4 == nr and (q + 1) % 4 == 1
    mesh = plsc.VectorSubcoreMesh(core_axis_name="c", subcore_axis_name="s", num_cores=NC, num_subcores=NS)

    nb = 4
    nloops = q // nb
    assert q % nb == 0

    @functools.partial(
        pl.kernel,
        out_type=jax.ShapeDtypeStruct((NC, NS, npt, dh), jnp.float32),
        mesh=mesh,
        scratch_types=[
            pltpu.VMEM((q + 1, kc), jnp.int32),
            pltpu.VMEM((q + 1, kc), jnp.int32),
            [pltpu.VMEM((kc, dh), jnp.float32) for _ in range(nb)],
            pltpu.VMEM((NZ, dh), jnp.float32),
            [pltpu.SemaphoreType.DMA for _ in range(nb)],
            pltpu.VMEM_SHARED((n, dh), jnp.float32),
        ],
        compiler_params=pltpu.CompilerParams(use_tc_tiling_on_sc=False),
    )
    def scat_kernel(g_hbm, src_hbm, dst_hbm, out_hbm,
                    si_v, di_v, rows, z_v, sems, acc):
        cid = lax.axis_index("c")
        sid = lax.axis_index("s")
        _fill(z_v, NZ, dh, 0.0)
        for j in range(npt // NZ):
            pltpu.sync_copy(z_v, acc.at[pl.ds(sid * npt + j * NZ, NZ)])
        r0 = sid * q + jnp.maximum(sid - (NS - 4), 0)
        nt = jnp.where(sid >= NS - 4, q + 1, q)
        pltpu.sync_copy(src_hbm.at[pl.ds(r0, q + 1)], si_v)
        pltpu.sync_copy(dst_hbm.at[pl.ds(r0, q + 1)], di_v)
        plsc.subcore_barrier()

        for b in range(nb):
            pltpu.async_copy(g_hbm.at[cid].at[si_v.at[b]], rows[b], sems[b])

        def ebody(jj, carry):
            for b in range(nb):
                i = jj * nb + b
                pltpu.make_async_copy(g_hbm.at[cid].at[si_v.at[i]],
                                      rows[b], sems[b]).wait()
                pltpu.sync_copy(rows[b], acc.at[di_v.at[i]], add=True)

                @pl.when(i + nb < nt)
                def _():
                    pltpu.async_copy(g_hbm.at[cid].at[si_v.at[i + nb]],
                                     rows[b], sems[b])
            return carry

        lax.fori_loop(0, nloops, ebody, 0)

        @pl.when(nt > q)
        def _():
            pltpu.make_async_copy(g_hbm.at[cid].at[si_v.at[q]],
                                  rows[0], sems[0]).wait()
            pltpu.sync_copy(rows[0], acc.at[di_v.at[q]], add=True)

        plsc.subcore_barrier()
        pltpu.sync_copy(acc.at[pl.ds(sid * npt, npt)], out_hbm.at[cid, sid])

    return scat_kernel


def _make_tc_layer1(dh):
    def _tc_layer1(x_ref, w_ref, deg_ref, o_ref):
        dinv = lax.rsqrt(deg_ref[...] + 1.0)
        xw = jnp.dot(x_ref[...], w_ref[...],
                     preferred_element_type=jnp.float32) * dinv
        o_ref[0] = xw[:, :dh]
        o_ref[1] = xw[:, dh:]

    return _tc_layer1


def _make_tc_layer2(d2h):
    def _tc_layer2(p_ref, g_ref, deg_ref, w_ref, o_ref):
        dinv = lax.rsqrt(deg_ref[...] + 1.0)
        s = p_ref[...] + g_ref[...]
        h = jnp.concatenate([s[0], s[1]], axis=1) * dinv
        h = jnp.maximum(h, 0.0)
        g2 = jnp.dot(h, w_ref[...], preferred_element_type=jnp.float32) * dinv
        o_ref[0] = g2[:, :d2h]
        o_ref[1] = g2[:, d2h:]

    return _tc_layer2


def _make_tc_final(d_out):
    def _tc_final(p_ref, g_ref, deg_ref, o_ref):
        dinv = lax.rsqrt(deg_ref[...] + 1.0)
        s = p_ref[...] + g_ref[...]
        out = jnp.concatenate([s[0], s[1]], axis=1) * dinv
        o_ref[...] = out[:, :d_out]

    return _tc_final


def kernel(x, edge_index, y, W1, W2):
    n, _ = x.shape
    e = edge_index.shape[1]
    d_hid = W1.shape[1]
    d_out = W2.shape[1]
    d2p = 64
    dh = d_hid // 2
    d2h = d2p // 2

    ept = e // (NC * NS)
    srcd = edge_index[0].reshape(e // 128, 128)
    dstd = edge_index[1].reshape(e // 128, 128)
    dst_deg = edge_index[1].reshape(NC * NS, ept // K, K)

    degp = _make_deg(n, e)(dst_deg).reshape(NC, n, 16)
    deg = (degp[0] + degp[1])[:, :1]

    g1 = pl.pallas_call(
        _make_tc_layer1(dh),
        out_shape=jax.ShapeDtypeStruct((NC, n, dh), jnp.float32),
    )(x, W1, deg)

    p1 = _make_scatter(n, e, dh)(g1, srcd, dstd).reshape(NC, n, dh)

    w2p = jnp.pad(W2, ((0, 0), (0, d2p - d_out)))
    g2 = pl.pallas_call(
        _make_tc_layer2(d2h),
        out_shape=jax.ShapeDtypeStruct((NC, n, d2h), jnp.float32),
    )(p1, g1, deg, w2p)

    p2 = _make_scatter(n, e, d2h)(g2, srcd, dstd).reshape(NC, n, d2h)

    out = pl.pallas_call(
        _make_tc_final(d_out),
        out_shape=jax.ShapeDtypeStruct((n, d_out), jnp.float32),
    )(p2, g2, deg)
    return out
